# baseline (speedup 1.0000x reference)
import functools

import jax
import jax.numpy as jnp
from jax import lax
from jax.experimental import pallas as pl
from jax.experimental.pallas import tpu as pltpu
from jax.experimental.pallas import tpu_sc as plsc

NC, NS = 2, 16
NW = NC * NS
F = 128
N = 10000
NPAD = 10112
E = 320000
CHUNK = 64
EPW = 10240
NCHUNKS = EPW // CHUNK
EPAD = NW * EPW
ROWS_PT = NPAD // NS

NPAIRS = 4096
GTOT = 3 * NPAIRS
GCH_PW = GTOT // (NW * CHUNK)

_MESH = plsc.VectorSubcoreMesh(core_axis_name="c", subcore_axis_name="s")


NBUF = 4
PASSES = 4
PCHUNKS = NCHUNKS // PASSES
NGROUPS = PCHUNKS // NBUF


def _scatter_body(sup_hbm, srcw_hbm, dstw_hbm, zeros_hbm, out_hbm,
                  src_idx_v, dst_idx_v, rows_v, acc_sh, gsem, ssem):
    c = lax.axis_index("c")
    s = lax.axis_index("s")
    wid = c * NS + s

    pltpu.sync_copy(zeros_hbm.at[pl.ds(s * ROWS_PT, ROWS_PT)],
                    acc_sh.at[pl.ds(s * ROWS_PT, ROWS_PT)])
    plsc.subcore_barrier()

    def _gather(i, b):
        pltpu.async_copy(sup_hbm.at[src_idx_v.at[i]], rows_v.at[b], gsem.at[b])

    def _gather_wait(i, b):
        pltpu.make_async_copy(sup_hbm.at[src_idx_v.at[i]], rows_v.at[b],
                              gsem.at[b]).wait()

    def _scat(i, b):
        pltpu.async_copy(rows_v.at[b], acc_sh.at[dst_idx_v.at[i]], ssem.at[b],
                         add=True)

    def _scat_wait(i, b):
        pltpu.make_async_copy(rows_v.at[b], acc_sh.at[dst_idx_v.at[i]],
                              ssem.at[b]).wait()

    for p in range(PASSES):
        pltpu.sync_copy(
            srcw_hbm.at[pl.ds(wid * NCHUNKS + p * PCHUNKS, PCHUNKS)], src_idx_v)
        pltpu.sync_copy(
            dstw_hbm.at[pl.ds(wid * NCHUNKS + p * PCHUNKS, PCHUNKS)], dst_idx_v)
        for b in range(NBUF):
            _gather(b, b)

        def group(g, carry):
            base = g * NBUF
            for b in range(NBUF):
                _gather_wait(base + b, b)
                _scat(base + b, b)
            for b in range(NBUF):
                _scat_wait(base + b, b)
                _gather(base + NBUF + b, b)
            return carry

        lax.fori_loop(0, NGROUPS - 1, group, 0)
        base = (NGROUPS - 1) * NBUF
        for b in range(NBUF):
            _gather_wait(base + b, b)
            _scat(base + b, b)
        for b in range(NBUF):
            _scat_wait(base + b, b)

    plsc.subcore_barrier()
    pltpu.sync_copy(acc_sh.at[pl.ds(s * ROWS_PT, ROWS_PT)],
                    out_hbm.at[c, pl.ds(s * ROWS_PT, ROWS_PT)])


def _sc_scatter(sup, srcw, dstw, zeros):
    return pl.kernel(
        _scatter_body,
        out_type=jax.ShapeDtypeStruct((NC, NPAD, F), jnp.float32),
        mesh=_MESH,
        scratch_types=[
            pltpu.VMEM((PCHUNKS, CHUNK), jnp.int32),
            pltpu.VMEM((PCHUNKS, CHUNK), jnp.int32),
            pltpu.VMEM((NBUF, CHUNK, F), jnp.float32),
            pltpu.VMEM_SHARED((NPAD, F), jnp.float32),
            pltpu.SemaphoreType.DMA((NBUF,)),
            pltpu.SemaphoreType.DMA((NBUF,)),
        ],
    )(sup, srcw, dstw, zeros)


def _gather_body(tab_hbm, idxw_hbm, out_hbm, idx_v, rows_v):
    c = lax.axis_index("c")
    s = lax.axis_index("s")
    wid = c * NS + s
    pltpu.sync_copy(idxw_hbm.at[pl.ds(wid * GCH_PW * CHUNK, GCH_PW * CHUNK)], idx_v)
    for i in range(GCH_PW):
        pltpu.sync_copy(tab_hbm.at[idx_v.at[pl.ds(i * CHUNK, CHUNK)]], rows_v)
        pltpu.sync_copy(rows_v, out_hbm.at[pl.ds((wid * GCH_PW + i) * CHUNK, CHUNK)])


def _sc_gather(tab, idxw):
    return pl.kernel(
        _gather_body,
        out_type=jax.ShapeDtypeStruct((GTOT, F), jnp.float32),
        mesh=_MESH,
        scratch_types=[
            pltpu.VMEM((GCH_PW * CHUNK,), jnp.int32),
            pltpu.VMEM((CHUNK, F), jnp.float32),
        ],
    )(tab, idxw)


def _mm1_body(x_ref, w_ref, o_ref):
    o_ref[...] = jnp.dot(
        x_ref[...], w_ref[...], preferred_element_type=jnp.float32)


def _tc_mm1(x, W1):
    return pl.pallas_call(
        _mm1_body,
        grid=(10,),
        in_specs=[
            pl.BlockSpec((N // 10, F), lambda i: (i, 0)),
            pl.BlockSpec((F, F), lambda i: (0, 0)),
        ],
        out_specs=pl.BlockSpec((N // 10, F), lambda i: (i, 0)),
        out_shape=jax.ShapeDtypeStruct((N, F), jnp.float32),
    )(x, W1)


def _mm2_body(a_ref, b_ref, w_ref, o_ref):
    h = jnp.maximum(a_ref[0] + a_ref[1] + b_ref[...], 0.0)
    o_ref[...] = jnp.dot(h, w_ref[...], preferred_element_type=jnp.float32)


def _tc_relu_mm2(acc1, b1, W2):
    return pl.pallas_call(
        _mm2_body,
        grid=(10,),
        in_specs=[
            pl.BlockSpec((2, N // 10, F), lambda i: (0, i, 0)),
            pl.BlockSpec((1, F), lambda i: (0, 0)),
            pl.BlockSpec((F, F), lambda i: (0, 0)),
        ],
        out_specs=pl.BlockSpec((N // 10, F), lambda i: (i, 0)),
        out_shape=jax.ShapeDtypeStruct((N, F), jnp.float32),
    )(acc1, b1.reshape(1, F), W2)


def _pairsum_body(a_ref, b_ref, o_ref):
    o_ref[...] = a_ref[0] + a_ref[1] + b_ref[...]


def _tc_pairsum(acc2, b2):
    return pl.pallas_call(
        _pairsum_body,
        grid=(10,),
        in_specs=[
            pl.BlockSpec((2, N // 10, F), lambda i: (0, i, 0)),
            pl.BlockSpec((1, F), lambda i: (0, 0)),
        ],
        out_specs=pl.BlockSpec((N // 10, F), lambda i: (i, 0)),
        out_shape=jax.ShapeDtypeStruct((N, F), jnp.float32),
    )(acc2, b2.reshape(1, F))


def _loss_body(g_ref, o_ref):
    g = g_ref[...]
    gh = g[0:NPAIRS]
    gp = g[NPAIRS:2 * NPAIRS]
    gn = g[2 * NPAIRS:3 * NPAIRS]
    pos = jnp.sum(gh * gp, axis=1)
    neg = jnp.sum(gh * gn, axis=1)
    z = pos - neg
    l = jnp.maximum(-z, 0.0) + jnp.log1p(jnp.exp(-jnp.abs(z)))
    o_ref[...] = jnp.mean(l).reshape(1, 1)


def _tc_loss(g):
    return pl.pallas_call(
        _loss_body,
        out_shape=jax.ShapeDtypeStruct((1, 1), jnp.float32),
    )(g)


def kernel(x, adj, head, pos_tail, neg_tail, W1, b1, W2, b2):
    src = adj[0].astype(jnp.int32)
    dst = adj[1].astype(jnp.int32)
    pad = EPAD - E
    srcw = jnp.concatenate([src, jnp.zeros((pad,), jnp.int32)]).reshape(
        NW * NCHUNKS, CHUNK)
    dstw = jnp.concatenate([dst, jnp.full((pad,), N, jnp.int32)]).reshape(
        NW * NCHUNKS, CHUNK)
    zeros = jnp.zeros((NPAD, F), jnp.float32)
    gidx = jnp.concatenate([head, pos_tail, neg_tail]).astype(jnp.int32)

    s1 = _tc_mm1(x, W1)
    acc1 = _sc_scatter(s1, srcw, dstw, zeros)
    s2 = _tc_relu_mm2(acc1, b1, W2)
    acc2 = _sc_scatter(s2, srcw, dstw, zeros)
    h2 = _tc_pairsum(acc2, b2)
    g = _sc_gather(h2, gidx)
    loss = _tc_loss(g)
    return loss[0, 0]


# speedup vs baseline: 1.1129x; 1.1129x over previous
import functools

import jax
import jax.numpy as jnp
from jax import lax
from jax.experimental import pallas as pl
from jax.experimental.pallas import tpu as pltpu
from jax.experimental.pallas import tpu_sc as plsc

NC, NS = 2, 16
NW = NC * NS
F = 128
N = 10000
NPAD = 10112
E = 320000
CHUNK = 128
EPW = 10240
NCHUNKS = EPW // CHUNK
EPAD = NW * EPW
ROWS_PT = NPAD // NS

NPAIRS = 4096
GTOT = 3 * NPAIRS
GCH_PW = GTOT // (NW * CHUNK)

_MESH = plsc.VectorSubcoreMesh(core_axis_name="c", subcore_axis_name="s")


NBUF = 2
PASSES = 2
PCHUNKS = NCHUNKS // PASSES


def _scatter_body(sup_hbm, srcw_hbm, dstw_hbm, zeros_hbm, out_hbm,
                  src_idx_v, dst_idx_v, rows_v, acc_sh, gsem, ssem):
    c = lax.axis_index("c")
    s = lax.axis_index("s")
    wid = c * NS + s

    pltpu.sync_copy(zeros_hbm.at[pl.ds(s * ROWS_PT, ROWS_PT)],
                    acc_sh.at[pl.ds(s * ROWS_PT, ROWS_PT)])
    plsc.subcore_barrier()

    def _gather(i, b):
        pltpu.async_copy(sup_hbm.at[src_idx_v.at[i]], rows_v.at[b], gsem.at[b])

    def _gather_wait(i, b):
        pltpu.make_async_copy(sup_hbm.at[src_idx_v.at[i]], rows_v.at[b],
                              gsem.at[b]).wait()

    def _scat(i, b):
        pltpu.async_copy(rows_v.at[b], acc_sh.at[dst_idx_v.at[i]], ssem.at[b],
                         add=True)

    def _scat_wait(i, b):
        pltpu.make_async_copy(rows_v.at[b], acc_sh.at[dst_idx_v.at[i]],
                              ssem.at[b]).wait()

    for p in range(PASSES):
        pltpu.sync_copy(
            srcw_hbm.at[pl.ds(wid * NCHUNKS + p * PCHUNKS, PCHUNKS)], src_idx_v)
        pltpu.sync_copy(
            dstw_hbm.at[pl.ds(wid * NCHUNKS + p * PCHUNKS, PCHUNKS)], dst_idx_v)
        _gather(0, 0)
        _gather(1, 1)

        def step(i, b):
            _gather_wait(i, b)
            _scat(i, b)
            _scat_wait(i, b)
            _gather(i + NBUF, b)

        def pair(k, carry):
            step(2 * k, 0)
            step(2 * k + 1, 1)
            return carry

        lax.fori_loop(0, PCHUNKS // 2 - 1, pair, 0)
        base = PCHUNKS - 2
        for b in range(NBUF):
            _gather_wait(base + b, b)
            _scat(base + b, b)
        for b in range(NBUF):
            _scat_wait(base + b, b)

    plsc.subcore_barrier()
    pltpu.sync_copy(acc_sh.at[pl.ds(s * ROWS_PT, ROWS_PT)],
                    out_hbm.at[c, pl.ds(s * ROWS_PT, ROWS_PT)])


def _sc_scatter(sup, srcw, dstw, zeros):
    return pl.kernel(
        _scatter_body,
        out_type=jax.ShapeDtypeStruct((NC, NPAD, F), jnp.float32),
        mesh=_MESH,
        scratch_types=[
            pltpu.VMEM((PCHUNKS, CHUNK), jnp.int32),
            pltpu.VMEM((PCHUNKS, CHUNK), jnp.int32),
            pltpu.VMEM((NBUF, CHUNK, F), jnp.float32),
            pltpu.VMEM_SHARED((NPAD, F), jnp.float32),
            pltpu.SemaphoreType.DMA((NBUF,)),
            pltpu.SemaphoreType.DMA((NBUF,)),
        ],
    )(sup, srcw, dstw, zeros)


def _gather_body(tab_hbm, idxw_hbm, out_hbm, idx_v, rows_v):
    c = lax.axis_index("c")
    s = lax.axis_index("s")
    wid = c * NS + s
    pltpu.sync_copy(idxw_hbm.at[pl.ds(wid * GCH_PW * CHUNK, GCH_PW * CHUNK)], idx_v)
    for i in range(GCH_PW):
        pltpu.sync_copy(tab_hbm.at[idx_v.at[pl.ds(i * CHUNK, CHUNK)]], rows_v)
        pltpu.sync_copy(rows_v, out_hbm.at[pl.ds((wid * GCH_PW + i) * CHUNK, CHUNK)])


def _sc_gather(tab, idxw):
    return pl.kernel(
        _gather_body,
        out_type=jax.ShapeDtypeStruct((GTOT, F), jnp.float32),
        mesh=_MESH,
        scratch_types=[
            pltpu.VMEM((GCH_PW * CHUNK,), jnp.int32),
            pltpu.VMEM((CHUNK, F), jnp.float32),
        ],
    )(tab, idxw)


def _mm1_body(x_ref, w_ref, o_ref):
    o_ref[...] = jnp.dot(
        x_ref[...], w_ref[...], preferred_element_type=jnp.float32)


def _tc_mm1(x, W1):
    return pl.pallas_call(
        _mm1_body,
        grid=(10,),
        in_specs=[
            pl.BlockSpec((N // 10, F), lambda i: (i, 0)),
            pl.BlockSpec((F, F), lambda i: (0, 0)),
        ],
        out_specs=pl.BlockSpec((N // 10, F), lambda i: (i, 0)),
        out_shape=jax.ShapeDtypeStruct((N, F), jnp.float32),
    )(x, W1)


def _mm2_body(a_ref, b_ref, w_ref, o_ref):
    h = jnp.maximum(a_ref[0] + a_ref[1] + b_ref[...], 0.0)
    o_ref[...] = jnp.dot(h, w_ref[...], preferred_element_type=jnp.float32)


def _tc_relu_mm2(acc1, b1, W2):
    return pl.pallas_call(
        _mm2_body,
        grid=(10,),
        in_specs=[
            pl.BlockSpec((2, N // 10, F), lambda i: (0, i, 0)),
            pl.BlockSpec((1, F), lambda i: (0, 0)),
            pl.BlockSpec((F, F), lambda i: (0, 0)),
        ],
        out_specs=pl.BlockSpec((N // 10, F), lambda i: (i, 0)),
        out_shape=jax.ShapeDtypeStruct((N, F), jnp.float32),
    )(acc1, b1.reshape(1, F), W2)


def _pairsum_body(a_ref, b_ref, o_ref):
    o_ref[...] = a_ref[0] + a_ref[1] + b_ref[...]


def _tc_pairsum(acc2, b2):
    return pl.pallas_call(
        _pairsum_body,
        grid=(10,),
        in_specs=[
            pl.BlockSpec((2, N // 10, F), lambda i: (0, i, 0)),
            pl.BlockSpec((1, F), lambda i: (0, 0)),
        ],
        out_specs=pl.BlockSpec((N // 10, F), lambda i: (i, 0)),
        out_shape=jax.ShapeDtypeStruct((N, F), jnp.float32),
    )(acc2, b2.reshape(1, F))


def _loss_body(g_ref, o_ref):
    g = g_ref[...]
    gh = g[0:NPAIRS]
    gp = g[NPAIRS:2 * NPAIRS]
    gn = g[2 * NPAIRS:3 * NPAIRS]
    pos = jnp.sum(gh * gp, axis=1)
    neg = jnp.sum(gh * gn, axis=1)
    z = pos - neg
    l = jnp.maximum(-z, 0.0) + jnp.log1p(jnp.exp(-jnp.abs(z)))
    o_ref[...] = jnp.mean(l).reshape(1, 1)


def _tc_loss(g):
    return pl.pallas_call(
        _loss_body,
        out_shape=jax.ShapeDtypeStruct((1, 1), jnp.float32),
    )(g)


def kernel(x, adj, head, pos_tail, neg_tail, W1, b1, W2, b2):
    src = adj[0].astype(jnp.int32)
    dst = adj[1].astype(jnp.int32)
    pad = EPAD - E
    srcw = jnp.concatenate([src, jnp.zeros((pad,), jnp.int32)]).reshape(
        NW * NCHUNKS, CHUNK)
    dstw = jnp.concatenate([dst, jnp.full((pad,), N, jnp.int32)]).reshape(
        NW * NCHUNKS, CHUNK)
    zeros = jnp.zeros((NPAD, F), jnp.float32)
    gidx = jnp.concatenate([head, pos_tail, neg_tail]).astype(jnp.int32)

    s1 = _tc_mm1(x, W1)
    acc1 = _sc_scatter(s1, srcw, dstw, zeros)
    s2 = _tc_relu_mm2(acc1, b1, W2)
    acc2 = _sc_scatter(s2, srcw, dstw, zeros)
    h2 = _tc_pairsum(acc2, b2)
    g = _sc_gather(h2, gidx)
    loss = _tc_loss(g)
    return loss[0, 0]
